# baseline (speedup 1.0000x reference)
import functools

import jax
import jax.numpy as jnp
from jax import lax
from jax.experimental import pallas as pl
from jax.experimental.pallas import tpu as pltpu
from jax.experimental.pallas import tpu_sc as plsc

_NC = 2
_NS = 16
_NW = _NC * _NS
_L = 16


def _pool_sc(x2d, idx5f, w_b, *, n_chunk):
    BT, C = x2d.shape
    BK = w_b.shape[0]
    rows_per_w = BK // _NW
    chunks = rows_per_w // n_chunk
    assert rows_per_w % n_chunk == 0

    mesh = plsc.VectorSubcoreMesh(
        core_axis_name="c", subcore_axis_name="s", num_cores=_NC, num_subcores=_NS
    )

    @functools.partial(
        pl.kernel,
        out_type=jax.ShapeDtypeStruct((BK, C), jnp.float32),
        mesh=mesh,
        scratch_types=[
            pltpu.VMEM((5 * n_chunk,), jnp.int32),
            pltpu.VMEM((5 * n_chunk, C), jnp.float32),
            pltpu.VMEM((n_chunk, C), jnp.float32),
            pltpu.VMEM((n_chunk, 5, _L), jnp.float32),
            pltpu.SemaphoreType.DMA,
        ],
    )
    def pool_kernel(x_hbm, idx_hbm, w_hbm, out_hbm, idx_v, buf_v, out_v, w_v, sem):
        wid = lax.axis_index("s") * _NC + lax.axis_index("c")
        for cc in range(chunks):
            base = wid * rows_per_w + cc * n_chunk
            pltpu.sync_copy(idx_hbm.at[pl.ds(base * 5, n_chunk * 5)], idx_v)
            pltpu.sync_copy(w_hbm.at[pl.ds(base, n_chunk)], w_v)
            pltpu.async_copy(x_hbm.at[idx_v], buf_v, sem).wait()

            def k_body(kk, _):
                w0 = w_v[kk, 0]
                w1 = w_v[kk, 1]
                w2 = w_v[kk, 2]
                w3 = w_v[kk, 3]
                w4 = w_v[kk, 4]

                def c_body(c, _):
                    for j in range(4):
                        s = pl.ds((c * 4 + j) * _L, _L)
                        r = w0 * buf_v[kk * 5 + 0, s]
                        r += w1 * buf_v[kk * 5 + 1, s]
                        r += w2 * buf_v[kk * 5 + 2, s]
                        r += w3 * buf_v[kk * 5 + 3, s]
                        r += w4 * buf_v[kk * 5 + 4, s]
                        out_v[kk, s] = r
                    return 0

                lax.fori_loop(0, C // (4 * _L), c_body, 0)
                return 0

            lax.fori_loop(0, n_chunk, k_body, 0)
            pltpu.sync_copy(out_v, out_hbm.at[pl.ds(base, n_chunk)])

    return pool_kernel(x2d, idx5f, w_b)


def _mlp_body(a_ref, w1_ref, b1_ref, w2_ref, b2_ref, o_ref):
    j = pl.program_id(1)
    h = jnp.dot(a_ref[...], w1_ref[...], preferred_element_type=jnp.float32)
    h = jnp.maximum(h + b1_ref[...], 0.0)
    p = jnp.dot(h, w2_ref[...], preferred_element_type=jnp.float32)

    @pl.when(j == 0)
    def _():
        o_ref[...] = p + b2_ref[...]

    @pl.when(j > 0)
    def _():
        o_ref[...] += p


def _mlp_tc(pooled, W1, b1, W2, b2, *, bm, bh):
    M, C = pooled.shape
    H = W1.shape[1]
    grid = (M // bm, H // bh)
    return pl.pallas_call(
        _mlp_body,
        grid=grid,
        in_specs=[
            pl.BlockSpec((bm, C), lambda i, j: (i, 0)),
            pl.BlockSpec((C, bh), lambda i, j: (0, j)),
            pl.BlockSpec((bh,), lambda i, j: (j,)),
            pl.BlockSpec((bh, 1), lambda i, j: (j, 0)),
            pl.BlockSpec((1,), lambda i, j: (0,)),
        ],
        out_specs=pl.BlockSpec((bm, 1), lambda i, j: (i, 0)),
        out_shape=jax.ShapeDtypeStruct((M, 1), jnp.float32),
        compiler_params=pltpu.CompilerParams(
            dimension_semantics=("parallel", "arbitrary"),
        ),
    )(pooled, W1, b1, W2, b2)


def kernel(x, coarse_ids, W1, b1, W2, b2):
    B, T, C = x.shape
    K = coarse_ids.shape[1]
    BK = B * K

    ids = coarse_ids.reshape(BK).astype(jnp.int32)
    boff = (jnp.arange(BK, dtype=jnp.int32) // K) * T
    offs = jnp.arange(-2, 3, dtype=jnp.int32)
    pos = ids[:, None] + offs[None, :]
    valid = (pos >= 0) & (pos < T)
    posc = jnp.clip(pos, 0, T - 1)
    idx5f = (boff[:, None] + posc).reshape(BK * 5)
    count = valid.sum(axis=1).astype(jnp.float32)
    w = valid.astype(jnp.float32) / count[:, None]
    w_b = jnp.broadcast_to(w[:, :, None], (BK, 5, _L))

    pooled = _pool_sc(x.reshape(B * T, C), idx5f, w_b, n_chunk=16)
    out = _mlp_tc(
        pooled.astype(jnp.bfloat16), W1.astype(jnp.bfloat16), b1, W2, b2,
        bm=BK, bh=512,
    )
    return out.reshape(B, K)


# speedup vs baseline: 1.4835x; 1.4835x over previous
import functools

import jax
import jax.numpy as jnp
from jax import lax
from jax.experimental import pallas as pl
from jax.experimental.pallas import tpu as pltpu
from jax.experimental.pallas import tpu_sc as plsc

_NC = 2
_NS = 16
_NW = _NC * _NS
_L = 16


def _pool_sc(x2d, idx5f, w_b, *, n_chunk):
    BT, C = x2d.shape
    BK = w_b.shape[0]
    rows_per_w = BK // _NW
    chunks = rows_per_w // n_chunk
    assert rows_per_w % n_chunk == 0

    mesh = plsc.VectorSubcoreMesh(
        core_axis_name="c", subcore_axis_name="s", num_cores=_NC, num_subcores=_NS
    )

    vm = lambda shape, dt: pltpu.VMEM(shape, dt)

    @functools.partial(
        pl.kernel,
        out_type=jax.ShapeDtypeStruct((BK, C), jnp.float32),
        mesh=mesh,
        scratch_types=[
            [vm((5 * n_chunk,), jnp.int32) for _ in range(2)],
            [vm((5 * n_chunk, C), jnp.float32) for _ in range(2)],
            [vm((n_chunk, C), jnp.float32) for _ in range(2)],
            [vm((n_chunk, 5, _L), jnp.float32) for _ in range(2)],
            [pltpu.SemaphoreType.DMA for _ in range(2)],
            [pltpu.SemaphoreType.DMA for _ in range(2)],
        ],
    )
    def pool_kernel(x_hbm, idx_hbm, w_hbm, out_hbm, idx_v, buf_v, out_v, w_v, gsem, osem):
        wid = lax.axis_index("s") * _NC + lax.axis_index("c")

        def stage(cc, sl):
            base = wid * rows_per_w + cc * n_chunk
            pltpu.sync_copy(idx_hbm.at[pl.ds(base * 5, n_chunk * 5)], idx_v[sl])
            pltpu.sync_copy(w_hbm.at[pl.ds(base, n_chunk)], w_v[sl])
            return pltpu.async_copy(x_hbm.at[idx_v[sl]], buf_v[sl], gsem[sl])

        gcp = [stage(0, 0), None]
        ocp = [None, None]
        for cc in range(chunks):
            cur = cc & 1
            nxt = cur ^ 1
            if cc + 1 < chunks:
                gcp[nxt] = stage(cc + 1, nxt)
            gcp[cur].wait()
            if ocp[cur] is not None:
                ocp[cur].wait()
            buf = buf_v[cur]
            out = out_v[cur]
            for q in range(n_chunk):
                w0 = w_v[cur][q, 0]
                w1 = w_v[cur][q, 1]
                w2 = w_v[cur][q, 2]
                w3 = w_v[cur][q, 3]
                w4 = w_v[cur][q, 4]

                @plsc.parallel_loop(0, C // _L, step=1, unroll=4)
                def c_body(c):
                    s = pl.ds(c * _L, _L)
                    r = w0 * buf[5 * q + 0, s]
                    r += w1 * buf[5 * q + 1, s]
                    r += w2 * buf[5 * q + 2, s]
                    r += w3 * buf[5 * q + 3, s]
                    r += w4 * buf[5 * q + 4, s]
                    out[q, s] = r

            base = wid * rows_per_w + cc * n_chunk
            ocp[cur] = pltpu.async_copy(out, out_hbm.at[pl.ds(base, n_chunk)], osem[cur])
        for sl in range(2):
            if ocp[sl] is not None:
                ocp[sl].wait()

    return pool_kernel(x2d, idx5f, w_b)


def _mlp_body(a_ref, w1_ref, b1_ref, w2_ref, b2_ref, o_ref):
    j = pl.program_id(1)
    h = jnp.dot(a_ref[...], w1_ref[...], preferred_element_type=jnp.float32)
    h = jnp.maximum(h + b1_ref[...], 0.0)
    p = jnp.dot(h, w2_ref[...], preferred_element_type=jnp.float32)

    @pl.when(j == 0)
    def _():
        o_ref[...] = p + b2_ref[...]

    @pl.when(j > 0)
    def _():
        o_ref[...] += p


def _mlp_tc(pooled, W1, b1, W2, b2, *, bm, bh):
    M, C = pooled.shape
    H = W1.shape[1]
    grid = (M // bm, H // bh)
    return pl.pallas_call(
        _mlp_body,
        grid=grid,
        in_specs=[
            pl.BlockSpec((bm, C), lambda i, j: (i, 0)),
            pl.BlockSpec((C, bh), lambda i, j: (0, j)),
            pl.BlockSpec((bh,), lambda i, j: (j,)),
            pl.BlockSpec((bh, 1), lambda i, j: (j, 0)),
            pl.BlockSpec((1,), lambda i, j: (0,)),
        ],
        out_specs=pl.BlockSpec((bm, 1), lambda i, j: (i, 0)),
        out_shape=jax.ShapeDtypeStruct((M, 1), jnp.float32),
        compiler_params=pltpu.CompilerParams(
            dimension_semantics=("parallel", "arbitrary"),
        ),
    )(pooled, W1, b1, W2, b2)


def kernel(x, coarse_ids, W1, b1, W2, b2):
    B, T, C = x.shape
    K = coarse_ids.shape[1]
    BK = B * K

    ids = coarse_ids.reshape(BK).astype(jnp.int32)
    boff = (jnp.arange(BK, dtype=jnp.int32) // K) * T
    offs = jnp.arange(-2, 3, dtype=jnp.int32)
    pos = ids[:, None] + offs[None, :]
    valid = (pos >= 0) & (pos < T)
    posc = jnp.clip(pos, 0, T - 1)
    idx5f = (boff[:, None] + posc).reshape(BK * 5)
    count = valid.sum(axis=1).astype(jnp.float32)
    w = valid.astype(jnp.float32) / count[:, None]
    w_b = jnp.broadcast_to(w[:, :, None], (BK, 5, _L))

    pooled = _pool_sc(x.reshape(B * T, C), idx5f, w_b, n_chunk=8)
    out = _mlp_tc(pooled, W1, b1, W2, b2, bm=BK, bh=512)
    return out.reshape(B, K)
